# Initial kernel scaffold; baseline (speedup 1.0000x reference)
import jax
import jax.numpy as jnp
from jax.experimental import pallas as pl


def kernel(alpha, dMag, fZ_vals, kEZ_val, fZs, kEZs, alphas, int_times, grid):
    raise NotImplementedError("write your pallas kernel here")


# speedup vs baseline: 19.9012x; 19.9012x over previous
import functools

import jax
import jax.numpy as jnp
from jax import lax
from jax.experimental import pallas as pl
from jax.experimental.pallas import tpu as pltpu
from jax.experimental.pallas import tpu_sc as plsc

N_FZ, N_KEZ, N_ALPHA, N_TINT = 64, 8, 256, 128
NORB, NTIMES = 512, 256
NC, NS, L = 2, 16, 16
NW = NC * NS
TPW = NTIMES // NW
JG = N_TINT // L
QG = NORB // L


def _prep_body(pf_ref, kz_ref, alpha_ref, dmag_ref, fzb_ref,
               s_ref, d_ref, m_ref, fzk_ref):
    la0 = pf_ref[0]
    inv_la = pf_ref[1]
    amin = pf_ref[2]
    amax = pf_ref[3]
    lf0 = pf_ref[4]
    inv_lf = pf_ref[5]
    kz = kz_ref[0]

    a = alpha_ref[...]
    a_ind = (jnp.log10(a) - la0) * inv_la
    a0 = jnp.clip(a_ind.astype(jnp.int32), 0, N_ALPHA - 1)
    d_ref[...] = a_ind - a0.astype(jnp.float32)
    s_ref[...] = jnp.minimum(a0, N_ALPHA - 2)
    geom = (a >= amin) & (a <= amax)
    m_ref[...] = jnp.where(geom, dmag_ref[...], jnp.float32(jnp.inf))

    fz_ind = (jnp.log10(fzb_ref[...]) - lf0) * inv_lf
    fz0 = jnp.clip(jnp.floor(fz_ind).astype(jnp.int32) + 1, 0, N_FZ - 2)
    fzk_ref[...] = fz0 * N_KEZ + kz


_prep = pl.pallas_call(
    _prep_body,
    in_specs=[
        pl.BlockSpec(memory_space=pltpu.SMEM),
        pl.BlockSpec(memory_space=pltpu.SMEM),
        pl.BlockSpec(memory_space=pltpu.VMEM),
        pl.BlockSpec(memory_space=pltpu.VMEM),
        pl.BlockSpec(memory_space=pltpu.VMEM),
    ],
    out_specs=[
        pl.BlockSpec(memory_space=pltpu.VMEM),
        pl.BlockSpec(memory_space=pltpu.VMEM),
        pl.BlockSpec(memory_space=pltpu.VMEM),
        pl.BlockSpec(memory_space=pltpu.VMEM),
    ],
    out_shape=[
        jax.ShapeDtypeStruct((NTIMES, NORB), jnp.int32),
        jax.ShapeDtypeStruct((NTIMES, NORB), jnp.float32),
        jax.ShapeDtypeStruct((NTIMES, NORB), jnp.float32),
        jax.ShapeDtypeStruct((NTIMES, L), jnp.int32),
    ],
)


@functools.partial(
    pl.kernel,
    mesh=plsc.VectorSubcoreMesh(core_axis_name="c", subcore_axis_name="s"),
    out_type=jax.ShapeDtypeStruct((NTIMES * N_TINT,), jnp.float32),
    scratch_types=[
        pltpu.VMEM((TPW * NORB,), jnp.int32),
        pltpu.VMEM((TPW * NORB,), jnp.float32),
        pltpu.VMEM((TPW * NORB,), jnp.float32),
        pltpu.VMEM((TPW * L,), jnp.int32),
        pltpu.VMEM((N_ALPHA * N_TINT,), jnp.float32),
        pltpu.VMEM((TPW * N_TINT,), jnp.float32),
    ],
)
def _sc_main(s_hbm, d_hbm, m_hbm, fzk_hbm, table_hbm, out_hbm,
             s_v, d_v, m_v, fz_v, slab_v, outbuf_v):
    wid = lax.axis_index("s") * NC + lax.axis_index("c")
    t0 = wid * TPW
    pltpu.sync_copy(s_hbm.at[pl.ds(t0 * NORB, TPW * NORB)], s_v)
    pltpu.sync_copy(d_hbm.at[pl.ds(t0 * NORB, TPW * NORB)], d_v)
    pltpu.sync_copy(m_hbm.at[pl.ds(t0 * NORB, TPW * NORB)], m_v)
    pltpu.sync_copy(fzk_hbm.at[pl.ds(t0 * L, TPW * L)], fz_v)

    for it in range(TPW):
        fzk = fz_v[pl.ds(it * L, L)][0]
        pltpu.sync_copy(table_hbm.at[fzk], slab_v)

        def qbody(g, accs, it=it):
            qb = it * NORB + g * L
            sv = s_v[pl.ds(qb, L)]
            dv = d_v[pl.ds(qb, L)]
            mv = m_v[pl.ds(qb, L)]
            accs = list(accs)
            for l in range(L):
                off = sv[l] * N_TINT
                dd = dv[l]
                mm = mv[l]
                for j in range(JG):
                    v0 = slab_v[pl.ds(off + j * L, L)]
                    v1 = slab_v[pl.ds(off + N_TINT + j * L, L)]
                    dim = v0 + dd * (v1 - v0)
                    hit = jnp.where(mm < dim, jnp.float32(1.0),
                                    jnp.float32(0.0))
                    accs[j] = accs[j] + hit
            return tuple(accs)

        zeros = tuple(jnp.zeros((L,), jnp.float32) for _ in range(JG))
        accs = lax.fori_loop(0, QG, qbody, zeros)
        scale = jnp.float32(1.0 / NORB)
        for j in range(JG):
            outbuf_v[pl.ds(it * N_TINT + j * L, L)] = accs[j] * scale

    pltpu.sync_copy(outbuf_v, out_hbm.at[pl.ds(t0 * N_TINT, TPW * N_TINT)])


def kernel(alpha, dMag, fZ_vals, kEZ_val, fZs, kEZs, alphas, int_times, grid):
    la = jnp.log10(alphas[:2])
    lf = jnp.log10(fZs[:2])
    pf = jnp.stack([
        la[0], 1.0 / (la[1] - la[0]),
        alphas[0], alphas[-1],
        lf[0], 1.0 / (lf[1] - lf[0]),
    ]).astype(jnp.float32)
    kz = jnp.clip(jnp.searchsorted(kEZs, kEZ_val, side="right") - 1,
                  0, N_KEZ - 1).astype(jnp.int32).reshape((1,))

    alpha_t = alpha.T.astype(jnp.float32)
    dmag_t = dMag.T.astype(jnp.float32)
    fzb = jnp.broadcast_to(fZ_vals.astype(jnp.float32)[:, None], (NTIMES, L))

    s, d, m, fzk = _prep(pf, kz, alpha_t, dmag_t, fzb)
    table = grid.reshape(N_FZ * N_KEZ, N_ALPHA * N_TINT)
    out = _sc_main(s.reshape(-1), d.reshape(-1), m.reshape(-1),
                   fzk.reshape(-1), table)
    return out.reshape(NTIMES, N_TINT)


# speedup vs baseline: 22.3083x; 1.1210x over previous
import functools

import jax
import jax.numpy as jnp
from jax import lax
from jax.experimental import pallas as pl
from jax.experimental.pallas import tpu as pltpu
from jax.experimental.pallas import tpu_sc as plsc

N_FZ, N_KEZ, N_ALPHA, N_TINT = 64, 8, 256, 128
NORB, NTIMES = 512, 256
NC, NS, L = 2, 16, 16
NW = NC * NS
TPW = NTIMES // NW
JG = N_TINT // L
QG = NORB // L


def _prep_body(pf_ref, kz_ref, alpha_ref, dmag_ref, fzb_ref,
               s_ref, d_ref, m_ref, fzk_ref):
    la0 = pf_ref[0]
    inv_la = pf_ref[1]
    amin = pf_ref[2]
    amax = pf_ref[3]
    lf0 = pf_ref[4]
    inv_lf = pf_ref[5]
    kz = kz_ref[0]

    a = alpha_ref[...]
    a_ind = (jnp.log10(a) - la0) * inv_la
    a0 = jnp.clip(a_ind.astype(jnp.int32), 0, N_ALPHA - 1)
    d_ref[...] = a_ind - a0.astype(jnp.float32)
    s_ref[...] = jnp.minimum(a0, N_ALPHA - 2)
    geom = (a >= amin) & (a <= amax)
    m_ref[...] = jnp.where(geom, dmag_ref[...], jnp.float32(jnp.inf))

    fz_ind = (jnp.log10(fzb_ref[...]) - lf0) * inv_lf
    fz0 = jnp.clip(jnp.floor(fz_ind).astype(jnp.int32) + 1, 0, N_FZ - 2)
    fzk_ref[...] = fz0 * N_KEZ + kz


_prep = pl.pallas_call(
    _prep_body,
    in_specs=[
        pl.BlockSpec(memory_space=pltpu.SMEM),
        pl.BlockSpec(memory_space=pltpu.SMEM),
        pl.BlockSpec(memory_space=pltpu.VMEM),
        pl.BlockSpec(memory_space=pltpu.VMEM),
        pl.BlockSpec(memory_space=pltpu.VMEM),
    ],
    out_specs=[
        pl.BlockSpec(memory_space=pltpu.VMEM),
        pl.BlockSpec(memory_space=pltpu.VMEM),
        pl.BlockSpec(memory_space=pltpu.VMEM),
        pl.BlockSpec(memory_space=pltpu.VMEM),
    ],
    out_shape=[
        jax.ShapeDtypeStruct((NTIMES, NORB), jnp.int32),
        jax.ShapeDtypeStruct((NTIMES, NORB), jnp.float32),
        jax.ShapeDtypeStruct((NTIMES, NORB), jnp.float32),
        jax.ShapeDtypeStruct((NTIMES, L), jnp.int32),
    ],
)


@functools.partial(
    pl.kernel,
    mesh=plsc.VectorSubcoreMesh(core_axis_name="c", subcore_axis_name="s"),
    out_type=jax.ShapeDtypeStruct((NTIMES * N_TINT,), jnp.float32),
    scratch_types=[
        pltpu.VMEM((TPW * NORB,), jnp.int32),
        pltpu.VMEM((TPW * NORB,), jnp.float32),
        pltpu.VMEM((TPW * NORB,), jnp.float32),
        pltpu.VMEM((TPW * L,), jnp.int32),
        pltpu.VMEM((2 * N_ALPHA * N_TINT,), jnp.float32),
        pltpu.VMEM((TPW * N_TINT,), jnp.float32),
        pltpu.SemaphoreType.DMA,
        pltpu.SemaphoreType.DMA,
    ],
)
def _sc_main(s_hbm, d_hbm, m_hbm, fzk_hbm, table_hbm, out_hbm,
             s_v, d_v, m_v, fz_v, slab_v, outbuf_v, sem0, sem1):
    wid = lax.axis_index("s") * NC + lax.axis_index("c")
    t0 = wid * TPW
    pltpu.sync_copy(fzk_hbm.at[pl.ds(t0 * L, TPW * L)], fz_v)
    sems = (sem0, sem1)
    SLAB = N_ALPHA * N_TINT

    def start_slab(it, b):
        fzk = fz_v[pl.ds(it * L, L)][0]
        return pltpu.async_copy(table_hbm.at[fzk],
                                slab_v.at[pl.ds(b * SLAB, SLAB)], sems[b])

    cds = [start_slab(0, 0), None]
    pltpu.sync_copy(s_hbm.at[pl.ds(t0 * NORB, TPW * NORB)], s_v)
    pltpu.sync_copy(d_hbm.at[pl.ds(t0 * NORB, TPW * NORB)], d_v)
    pltpu.sync_copy(m_hbm.at[pl.ds(t0 * NORB, TPW * NORB)], m_v)

    for it in range(TPW):
        b = it % 2
        cds[b].wait()
        if it + 1 < TPW:
            cds[(it + 1) % 2] = start_slab(it + 1, (it + 1) % 2)

        def qbody(g, accs, it=it, b=b):
            qb = it * NORB + g * L
            sv = s_v[pl.ds(qb, L)]
            dv = d_v[pl.ds(qb, L)]
            mv = m_v[pl.ds(qb, L)]
            accs = list(accs)
            for l in range(L):
                off = sv[l] * N_TINT + b * SLAB
                dd = dv[l]
                mm = mv[l]
                for j in range(JG):
                    v0 = slab_v[pl.ds(off + j * L, L)]
                    v1 = slab_v[pl.ds(off + N_TINT + j * L, L)]
                    dim = v0 + dd * (v1 - v0)
                    hit = jnp.where(mm < dim, jnp.float32(1.0),
                                    jnp.float32(0.0))
                    accs[j] = accs[j] + hit
            return tuple(accs)

        zeros = tuple(jnp.zeros((L,), jnp.float32) for _ in range(JG))
        accs = lax.fori_loop(0, QG, qbody, zeros)
        scale = jnp.float32(1.0 / NORB)
        for j in range(JG):
            outbuf_v[pl.ds(it * N_TINT + j * L, L)] = accs[j] * scale

    pltpu.sync_copy(outbuf_v, out_hbm.at[pl.ds(t0 * N_TINT, TPW * N_TINT)])


def kernel(alpha, dMag, fZ_vals, kEZ_val, fZs, kEZs, alphas, int_times, grid):
    la = jnp.log10(alphas[:2])
    lf = jnp.log10(fZs[:2])
    pf = jnp.stack([
        la[0], 1.0 / (la[1] - la[0]),
        alphas[0], alphas[-1],
        lf[0], 1.0 / (lf[1] - lf[0]),
    ]).astype(jnp.float32)
    kz = jnp.clip(jnp.searchsorted(kEZs, kEZ_val, side="right") - 1,
                  0, N_KEZ - 1).astype(jnp.int32).reshape((1,))

    alpha_t = alpha.T.astype(jnp.float32)
    dmag_t = dMag.T.astype(jnp.float32)
    fzb = jnp.broadcast_to(fZ_vals.astype(jnp.float32)[:, None], (NTIMES, L))

    s, d, m, fzk = _prep(pf, kz, alpha_t, dmag_t, fzb)
    table = grid.reshape(N_FZ * N_KEZ, N_ALPHA * N_TINT)
    out = _sc_main(s.reshape(-1), d.reshape(-1), m.reshape(-1),
                   fzk.reshape(-1), table)
    return out.reshape(NTIMES, N_TINT)


# speedup vs baseline: 24.8817x; 1.1154x over previous
import functools

import jax
import jax.numpy as jnp
from jax import lax
from jax.experimental import pallas as pl
from jax.experimental.pallas import tpu as pltpu
from jax.experimental.pallas import tpu_sc as plsc

N_FZ, N_KEZ, N_ALPHA, N_TINT = 64, 8, 256, 128
NORB, NTIMES = 512, 256
NC, NS, L = 2, 16, 16
NW = NC * NS
TPW = NTIMES // NW
JG = N_TINT // L
QG = NORB // L


def _prep_body(pf_ref, kz_ref, alpha_ref, dmag_ref, fzb_ref,
               s_ref, d_ref, m_ref, fzk_ref):
    la0 = pf_ref[0]
    inv_la = pf_ref[1]
    amin = pf_ref[2]
    amax = pf_ref[3]
    lf0 = pf_ref[4]
    inv_lf = pf_ref[5]
    kz = kz_ref[0]

    a = alpha_ref[...]
    a_ind = (jnp.log10(a) - la0) * inv_la
    a0 = jnp.clip(a_ind.astype(jnp.int32), 0, N_ALPHA - 1)
    d_ref[...] = a_ind - a0.astype(jnp.float32)
    s_ref[...] = jnp.minimum(a0, N_ALPHA - 2)
    geom = (a >= amin) & (a <= amax)
    m_ref[...] = jnp.where(geom, dmag_ref[...], jnp.float32(jnp.inf))

    del kz
    fz_ind = (jnp.log10(fzb_ref[...]) - lf0) * inv_lf
    fzk_ref[...] = jnp.clip(jnp.floor(fz_ind).astype(jnp.int32) + 1,
                            0, N_FZ - 2)


_prep = pl.pallas_call(
    _prep_body,
    in_specs=[
        pl.BlockSpec(memory_space=pltpu.SMEM),
        pl.BlockSpec(memory_space=pltpu.SMEM),
        pl.BlockSpec(memory_space=pltpu.VMEM),
        pl.BlockSpec(memory_space=pltpu.VMEM),
        pl.BlockSpec(memory_space=pltpu.VMEM),
    ],
    out_specs=[
        pl.BlockSpec(memory_space=pltpu.VMEM),
        pl.BlockSpec(memory_space=pltpu.VMEM),
        pl.BlockSpec(memory_space=pltpu.VMEM),
        pl.BlockSpec(memory_space=pltpu.VMEM),
    ],
    out_shape=[
        jax.ShapeDtypeStruct((NTIMES, NORB), jnp.int32),
        jax.ShapeDtypeStruct((NTIMES, NORB), jnp.float32),
        jax.ShapeDtypeStruct((NTIMES, NORB), jnp.float32),
        jax.ShapeDtypeStruct((NTIMES, L), jnp.int32),
    ],
)


@functools.partial(
    pl.kernel,
    mesh=plsc.VectorSubcoreMesh(core_axis_name="c", subcore_axis_name="s"),
    out_type=jax.ShapeDtypeStruct((NTIMES * N_TINT,), jnp.float32),
    scratch_types=[
        pltpu.VMEM((TPW * NORB,), jnp.int32),
        pltpu.VMEM((TPW * NORB,), jnp.float32),
        pltpu.VMEM((TPW * NORB,), jnp.float32),
        pltpu.VMEM((TPW * L,), jnp.int32),
        pltpu.VMEM((2 * N_ALPHA * N_TINT,), jnp.float32),
        pltpu.VMEM((TPW * N_TINT,), jnp.float32),
        pltpu.SemaphoreType.DMA,
        pltpu.SemaphoreType.DMA,
    ],
)
def _sc_main(s_hbm, d_hbm, m_hbm, fzk_hbm, table_hbm, out_hbm,
             s_v, d_v, m_v, fz_v, slab_v, outbuf_v, sem0, sem1):
    wid = lax.axis_index("s") * NC + lax.axis_index("c")
    t0 = wid * TPW
    pltpu.sync_copy(fzk_hbm.at[pl.ds(t0 * L, TPW * L)], fz_v)
    sems = (sem0, sem1)
    SLAB = N_ALPHA * N_TINT

    def start_slab(it, b):
        fzk = fz_v[pl.ds(it * L, L)][0]
        return pltpu.async_copy(table_hbm.at[fzk],
                                slab_v.at[pl.ds(b * SLAB, SLAB)], sems[b])

    cds = [start_slab(0, 0), None]
    pltpu.sync_copy(s_hbm.at[pl.ds(t0 * NORB, TPW * NORB)], s_v)
    pltpu.sync_copy(d_hbm.at[pl.ds(t0 * NORB, TPW * NORB)], d_v)
    pltpu.sync_copy(m_hbm.at[pl.ds(t0 * NORB, TPW * NORB)], m_v)

    for it in range(TPW):
        b = it % 2
        cds[b].wait()
        if it + 1 < TPW:
            cds[(it + 1) % 2] = start_slab(it + 1, (it + 1) % 2)

        def qbody(g, accs, it=it, b=b):
            qb = it * NORB + g * L
            sv = s_v[pl.ds(qb, L)]
            dv = d_v[pl.ds(qb, L)]
            mv = m_v[pl.ds(qb, L)]
            accs = list(accs)
            for l in range(L):
                off = sv[l] * N_TINT + b * SLAB
                dd = dv[l]
                mm = mv[l]
                for j in range(JG):
                    v0 = slab_v[pl.ds(off + j * L, L)]
                    v1 = slab_v[pl.ds(off + N_TINT + j * L, L)]
                    dim = v0 + dd * (v1 - v0)
                    hit = jnp.where(mm < dim, jnp.float32(1.0),
                                    jnp.float32(0.0))
                    accs[j] = accs[j] + hit
            return tuple(accs)

        zeros = tuple(jnp.zeros((L,), jnp.float32) for _ in range(JG))
        accs = lax.fori_loop(0, QG, qbody, zeros)
        scale = jnp.float32(1.0 / NORB)
        for j in range(JG):
            outbuf_v[pl.ds(it * N_TINT + j * L, L)] = accs[j] * scale

    pltpu.sync_copy(outbuf_v, out_hbm.at[pl.ds(t0 * N_TINT, TPW * N_TINT)])


def kernel(alpha, dMag, fZ_vals, kEZ_val, fZs, kEZs, alphas, int_times, grid):
    la = jnp.log10(alphas[:2])
    lf = jnp.log10(fZs[:2])
    pf = jnp.stack([
        la[0], 1.0 / (la[1] - la[0]),
        alphas[0], alphas[-1],
        lf[0], 1.0 / (lf[1] - lf[0]),
    ]).astype(jnp.float32)
    kz = jnp.clip(jnp.searchsorted(kEZs, kEZ_val, side="right") - 1,
                  0, N_KEZ - 1).astype(jnp.int32)

    alpha_t = alpha.T.astype(jnp.float32)
    dmag_t = dMag.T.astype(jnp.float32)
    fzb = jnp.broadcast_to(fZ_vals.astype(jnp.float32)[:, None], (NTIMES, L))

    s, d, m, fzk = _prep(pf, kz.reshape((1,)), alpha_t, dmag_t, fzb)
    table = lax.dynamic_index_in_dim(grid, kz, axis=1, keepdims=False)
    table = table.reshape(N_FZ, N_ALPHA * N_TINT)
    out = _sc_main(s.reshape(-1), d.reshape(-1), m.reshape(-1),
                   fzk.reshape(-1), table)
    return out.reshape(NTIMES, N_TINT)


# speedup vs baseline: 30.5316x; 1.2271x over previous
import functools

import jax
import jax.numpy as jnp
from jax import lax
from jax.experimental import pallas as pl
from jax.experimental.pallas import tpu as pltpu
from jax.experimental.pallas import tpu_sc as plsc

N_FZ, N_KEZ, N_ALPHA, N_TINT = 64, 8, 256, 128
NORB, NTIMES = 512, 256
NC, NS, L = 2, 16, 16
NW = NC * NS
TPW = NTIMES // NW
JG = N_TINT // L
QG = NORB // L


def _prep_body(pf_ref, alpha_ref, dmag_ref, fzb_ref,
               s_ref, d_ref, m_ref, fzk_ref):
    la0 = pf_ref[0]
    inv_la = 1.0 / (pf_ref[1] - pf_ref[0])
    lf0 = pf_ref[2]
    inv_lf = 1.0 / (pf_ref[3] - pf_ref[2])
    amin = pf_ref[4]
    amax = pf_ref[5]

    a = alpha_ref[...]
    a_ind = (jnp.log10(a) - la0) * inv_la
    a0 = jnp.clip(a_ind.astype(jnp.int32), 0, N_ALPHA - 1)
    d_ref[...] = jnp.swapaxes(a_ind - a0.astype(jnp.float32), 0, 1)
    s_ref[...] = jnp.swapaxes(jnp.minimum(a0, N_ALPHA - 2), 0, 1)
    geom = (a >= amin) & (a <= amax)
    m_ref[...] = jnp.swapaxes(
        jnp.where(geom, dmag_ref[...], jnp.float32(jnp.inf)), 0, 1)

    fz_ind = (jnp.log10(fzb_ref[...]) - lf0) * inv_lf
    fzk_ref[...] = jnp.clip(jnp.floor(fz_ind).astype(jnp.int32) + 1,
                            0, N_FZ - 2)


_prep = pl.pallas_call(
    _prep_body,
    in_specs=[
        pl.BlockSpec(memory_space=pltpu.SMEM),
        pl.BlockSpec(memory_space=pltpu.VMEM),
        pl.BlockSpec(memory_space=pltpu.VMEM),
        pl.BlockSpec(memory_space=pltpu.VMEM),
    ],
    out_specs=[
        pl.BlockSpec(memory_space=pltpu.VMEM),
        pl.BlockSpec(memory_space=pltpu.VMEM),
        pl.BlockSpec(memory_space=pltpu.VMEM),
        pl.BlockSpec(memory_space=pltpu.VMEM),
    ],
    out_shape=[
        jax.ShapeDtypeStruct((NTIMES, NORB), jnp.int32),
        jax.ShapeDtypeStruct((NTIMES, NORB), jnp.float32),
        jax.ShapeDtypeStruct((NTIMES, NORB), jnp.float32),
        jax.ShapeDtypeStruct((NTIMES, L), jnp.int32),
    ],
)


@functools.partial(
    pl.kernel,
    mesh=plsc.VectorSubcoreMesh(core_axis_name="c", subcore_axis_name="s"),
    out_type=jax.ShapeDtypeStruct((NTIMES, N_TINT), jnp.float32),
    scratch_types=[
        pltpu.VMEM((TPW, NORB), jnp.int32),
        pltpu.VMEM((TPW, NORB), jnp.float32),
        pltpu.VMEM((TPW, NORB), jnp.float32),
        pltpu.VMEM((TPW, L), jnp.int32),
        pltpu.VMEM((2 * N_ALPHA * N_TINT,), jnp.float32),
        pltpu.VMEM((TPW, N_TINT), jnp.float32),
        pltpu.SemaphoreType.DMA,
        pltpu.SemaphoreType.DMA,
    ],
)
def _sc_main(s_hbm, d_hbm, m_hbm, fzk_hbm, table_hbm, out_hbm,
             s_v, d_v, m_v, fz_v, slab_v, outbuf_v, sem0, sem1):
    wid = lax.axis_index("s") * NC + lax.axis_index("c")
    t0 = wid * TPW
    pltpu.sync_copy(fzk_hbm.at[pl.ds(t0, TPW)], fz_v)
    sems = (sem0, sem1)
    SLAB = N_ALPHA * N_TINT

    def start_slab(it, b):
        fzk = fz_v[it, pl.ds(0, L)][0]
        return pltpu.async_copy(table_hbm.at[fzk],
                                slab_v.at[pl.ds(b * SLAB, SLAB)], sems[b])

    cds = [start_slab(0, 0), None]
    pltpu.sync_copy(s_hbm.at[pl.ds(t0, TPW)], s_v)
    pltpu.sync_copy(d_hbm.at[pl.ds(t0, TPW)], d_v)
    pltpu.sync_copy(m_hbm.at[pl.ds(t0, TPW)], m_v)

    for it in range(TPW):
        b = it % 2
        cds[b].wait()
        if it + 1 < TPW:
            cds[(it + 1) % 2] = start_slab(it + 1, (it + 1) % 2)

        def qbody(g, accs, it=it, b=b):
            qb = g * L
            sv = s_v[it, pl.ds(qb, L)]
            dv = d_v[it, pl.ds(qb, L)]
            mv = m_v[it, pl.ds(qb, L)]
            accs = list(accs)
            for l in range(L):
                off = sv[l] * N_TINT + b * SLAB
                dd = dv[l]
                mm = mv[l]
                for j in range(JG):
                    v0 = slab_v[pl.ds(off + j * L, L)]
                    v1 = slab_v[pl.ds(off + N_TINT + j * L, L)]
                    dim = v0 + dd * (v1 - v0)
                    hit = jnp.where(mm < dim, jnp.float32(1.0),
                                    jnp.float32(0.0))
                    accs[j] = accs[j] + hit
            return tuple(accs)

        zeros = tuple(jnp.zeros((L,), jnp.float32) for _ in range(JG))
        accs = lax.fori_loop(0, QG, qbody, zeros)
        scale = jnp.float32(1.0 / NORB)
        for j in range(JG):
            outbuf_v[it, pl.ds(j * L, L)] = accs[j] * scale

    pltpu.sync_copy(outbuf_v, out_hbm.at[pl.ds(t0, TPW)])


def kernel(alpha, dMag, fZ_vals, kEZ_val, fZs, kEZs, alphas, int_times, grid):
    kz = jnp.clip(jnp.sum((kEZs <= kEZ_val).astype(jnp.int32)) - 1,
                  0, N_KEZ - 1)
    lg4 = jnp.log10(jnp.concatenate([alphas[:2], fZs[:2]]))
    pf = jnp.concatenate([lg4, alphas[:1], alphas[-1:]]).astype(jnp.float32)

    fzb = jnp.broadcast_to(fZ_vals.astype(jnp.float32)[:, None], (NTIMES, L))

    s, d, m, fzk = _prep(pf, alpha, dMag, fzb)
    table = lax.dynamic_index_in_dim(grid, kz, axis=1, keepdims=False)
    table = table.reshape(N_FZ, N_ALPHA * N_TINT)
    return _sc_main(s, d, m, fzk, table)
